# baseline (speedup 1.0000x reference)
import jax
import jax.numpy as jnp
from jax import lax
from jax.experimental import pallas as pl
from jax.experimental.pallas import tpu as pltpu
from jax.experimental.pallas import tpu_sc as plsc

N_COEF = 0.3

B, D = 4096, 2048
NC, NS, L = 2, 16, 16
NW = NC * NS
ROWS_PER_W = B // NW
CHUNK = 4
NCHUNK = ROWS_PER_W // CHUNK
NSLOT = 3
VECS = CHUNK * D // L
JSHIFT = (D // L).bit_length() - 1


def _sc_body(x_hbm, r_hbm, out_hbm, idx_v, gbuf, xbuf, obuf, sems):
    wid = lax.axis_index("s") * NC + lax.axis_index("c")
    base = wid * ROWS_PER_W

    pltpu.sync_copy(r_hbm.at[pl.ds(wid * NCHUNK, NCHUNK)], idx_v)

    def issue_in(c, s):
        pltpu.async_copy(
            x_hbm.at[idx_v.at[c]], gbuf.at[s], sems.at[s])
        pltpu.async_copy(
            x_hbm.at[pl.ds(base + c * CHUNK, CHUNK)], xbuf.at[s],
            sems.at[NSLOT + s])

    def wait_in(s):
        pltpu.make_async_copy(x_hbm.at[pl.ds(0, CHUNK)], gbuf.at[s],
                              sems.at[s]).wait()
        pltpu.make_async_copy(x_hbm.at[pl.ds(0, CHUNK)], xbuf.at[s],
                              sems.at[NSLOT + s]).wait()

    def issue_out(c, so):
        pltpu.async_copy(
            obuf.at[so], out_hbm.at[pl.ds(base + c * CHUNK, CHUNK)],
            sems.at[2 * NSLOT + so])

    def wait_out(so):
        pltpu.make_async_copy(obuf.at[so], out_hbm.at[pl.ds(0, CHUNK)],
                              sems.at[2 * NSLOT + so]).wait()

    for p in range(NSLOT):
        issue_in(p, p)

    def step(c, carry):
        s = lax.rem(c, NSLOT)
        so = c & 1
        wait_in(s)

        @pl.when(c >= 2)
        def _drain():
            wait_out(so)

        def blend(k, _):
            i = k >> JSHIFT
            j = (k - (i << JSHIFT)) * L
            g = gbuf[s, i, pl.ds(j, L)]
            xv = xbuf[s, i, pl.ds(j, L)]
            obuf[so, i, pl.ds(j, L)] = xv + N_COEF * (g - xv)
            return _

        lax.fori_loop(0, VECS, blend, 0, unroll=8)
        issue_out(c, so)

        @pl.when(c + NSLOT < NCHUNK)
        def _prefetch():
            issue_in(c + NSLOT, s)

        return carry

    lax.fori_loop(0, NCHUNK, step, 0)
    wait_out(NCHUNK & 1)
    wait_out((NCHUNK + 1) & 1)


@jax.jit
def _batchout(x, r):
    mesh = plsc.VectorSubcoreMesh(core_axis_name="c", subcore_axis_name="s")
    run = pl.kernel(
        _sc_body,
        out_type=jax.ShapeDtypeStruct((B, D), jnp.float32),
        mesh=mesh,
        scratch_types=[
            pltpu.VMEM((NCHUNK, CHUNK), jnp.int32),
            pltpu.VMEM((NSLOT, CHUNK, D), jnp.float32),
            pltpu.VMEM((NSLOT, CHUNK, D), jnp.float32),
            pltpu.VMEM((2, CHUNK, D), jnp.float32),
            pltpu.SemaphoreType.DMA((2 * NSLOT + 2,)),
        ],
    )
    return run(x, r)


def kernel(x, y, r):
    x_out = _batchout(x, r.reshape(B // CHUNK, CHUNK))
    return (x_out, r)


# speedup vs baseline: 1.0351x; 1.0351x over previous
import jax
import jax.numpy as jnp
from jax import lax
from jax.experimental import pallas as pl
from jax.experimental.pallas import tpu as pltpu
from jax.experimental.pallas import tpu_sc as plsc

N_COEF = 0.3

B, D = 4096, 2048
NC, NS, L = 2, 16, 16
NW = NC * NS
ROWS_PER_W = B // NW
GCHUNK = 16
NPAIR = ROWS_PER_W // GCHUNK
CHUNK = 8
VECS = CHUNK * D // L
JSHIFT = (D // L).bit_length() - 1


def _sc_body(x_hbm, r_hbm, out_hbm, idx_v, gbuf, xbuf, sems):
    wid = lax.axis_index("s") * NC + lax.axis_index("c")
    base = wid * ROWS_PER_W

    pltpu.sync_copy(r_hbm.at[pl.ds(wid * NPAIR, NPAIR)], idx_v)

    def issue_g(p, sp):
        pltpu.async_copy(x_hbm.at[idx_v.at[p]], gbuf.at[sp], sems.at[sp])

    def wait_g(sp):
        pltpu.make_async_copy(x_hbm.at[pl.ds(0, GCHUNK)], gbuf.at[sp],
                              sems.at[sp]).wait()

    def issue_x(c, sx):
        pltpu.async_copy(
            x_hbm.at[pl.ds(base + c * CHUNK, CHUNK)], xbuf.at[sx],
            sems.at[2 + sx])

    def wait_x(sx):
        pltpu.make_async_copy(x_hbm.at[pl.ds(0, CHUNK)], xbuf.at[sx],
                              sems.at[2 + sx]).wait()

    def issue_out(c, sx):
        pltpu.async_copy(
            xbuf.at[sx], out_hbm.at[pl.ds(base + c * CHUNK, CHUNK)],
            sems.at[4 + sx])

    def wait_out(sx):
        pltpu.make_async_copy(xbuf.at[sx], out_hbm.at[pl.ds(0, CHUNK)],
                              sems.at[4 + sx]).wait()

    def blend_chunk(sx, sp, roff):
        def blend(k, carry):
            i = k >> JSHIFT
            j = (k - (i << JSHIFT)) * L
            g = gbuf[sp, roff + i, pl.ds(j, L)]
            xv = xbuf[sx, i, pl.ds(j, L)]
            xbuf[sx, i, pl.ds(j, L)] = xv + N_COEF * (g - xv)
            return carry

        lax.fori_loop(0, VECS, blend, 0, unroll=8)

    issue_g(0, 0)
    issue_x(0, 0)
    issue_x(1, 1)

    def step(p, carry):
        sp = p & 1
        c0 = 2 * p

        @pl.when(p + 1 < NPAIR)
        def _pg():
            issue_g(p + 1, 1 - sp)

        wait_g(sp)

        sx0 = c0 & 1
        wait_x(sx0)

        @pl.when(c0 >= 2)
        def _d0():
            wait_out(sx0)

        blend_chunk(sx0, sp, 0)
        issue_out(c0, sx0)

        @pl.when(c0 + 2 < 2 * NPAIR)
        def _px0():
            issue_x(c0 + 2, sx0)

        sx1 = sx0 ^ 1
        wait_x(sx1)

        @pl.when(c0 + 1 >= 2)
        def _d1():
            wait_out(sx1)

        blend_chunk(sx1, sp, CHUNK)
        issue_out(c0 + 1, sx1)

        @pl.when(c0 + 3 < 2 * NPAIR)
        def _px1():
            issue_x(c0 + 3, sx1)

        return carry

    lax.fori_loop(0, NPAIR, step, 0)
    wait_out(0)
    wait_out(1)


@jax.jit
def _batchout(x, r2):
    mesh = plsc.VectorSubcoreMesh(core_axis_name="c", subcore_axis_name="s")
    run = pl.kernel(
        _sc_body,
        out_type=jax.ShapeDtypeStruct((B, D), jnp.float32),
        mesh=mesh,
        scratch_types=[
            pltpu.VMEM((NPAIR, GCHUNK), jnp.int32),
            pltpu.VMEM((2, GCHUNK, D), jnp.float32),
            pltpu.VMEM((2, CHUNK, D), jnp.float32),
            pltpu.SemaphoreType.DMA((6,)),
        ],
    )
    return run(x, r2)


def kernel(x, y, r):
    x_out = _batchout(x, r.reshape(B // GCHUNK, GCHUNK))
    return (x_out, r)
